# Initial kernel scaffold; baseline (speedup 1.0000x reference)
import jax
import jax.numpy as jnp
from jax.experimental import pallas as pl


def kernel(values, lengths, table, metadata):
    raise NotImplementedError("write your pallas kernel here")


# speedup vs baseline: 20.6340x; 20.6340x over previous
import functools

import jax
import jax.numpy as jnp
from jax import lax
from jax.experimental import pallas as pl
from jax.experimental.pallas import tpu as pltpu
from jax.experimental.pallas import tpu_sc as plsc

jax.config.update("jax_enable_x64", True)

ZCH = 1_000_000
PLANE = 1 << 20
EDIM = 32
NVAL = 327_680
NWORK = 32
PERW = NVAL // NWORK
CH = 128
NCH = PERW // CH
RING = 8

BW = 25
BCH = 4000
BLOOP = ZCH // (BW * BCH)

_K1_LO, _K1_HI = 0x7F4A7C15, 0x9E3779B9
_K2_LO, _K2_HI = 0x1CE4E5B9, 0xBF58476D
_M80 = (2**35 + 79) // 80
_M3125 = (2**42 + 3124) // 3125
_R32_12500 = 4796


def _u(x):
    return jnp.uint32(x)


def _mulhi(a, b_const):
    b0 = _u(b_const & 0xFFFF)
    b1 = _u(b_const >> 16)
    a0 = a & _u(0xFFFF)
    a1 = a >> _u(16)
    p00 = a0 * b0
    p01 = a0 * b1
    p10 = a1 * b0
    p11 = a1 * b1
    w = (p00 >> _u(16)) + (p01 & _u(0xFFFF)) + (p10 & _u(0xFFFF))
    return p11 + (p01 >> _u(16)) + (p10 >> _u(16)) + (w >> _u(16))


def _mod12500(x):
    q = _mulhi(x >> _u(2), _M3125) >> _u(10)
    return x - q * _u(12500)


def _remap16(v):
    qb = _mulhi(v, _M80) >> _u(3)
    bucket = v - qb * _u(80)
    lo1 = v * _u(_K1_LO)
    hi1 = _mulhi(v, _K1_LO) + v * _u(_K1_HI)
    lo2 = lo1 ^ ((hi1 << _u(3)) | (lo1 >> _u(29)))
    hi2 = hi1 ^ (hi1 >> _u(29))
    lo3 = lo2 * _u(_K2_LO)
    hi3 = _mulhi(lo2, _K2_LO) + lo2 * _u(_K2_HI) + hi2 * _u(_K2_LO)
    lo4 = lo3 ^ hi3
    s = _mod12500(_mod12500(hi3) * _u(_R32_12500) + _mod12500(lo4))
    return plsc.bitcast(bucket * _u(12500) + s, jnp.int32)


def _sc_body(v_hbm, table_hbm, emb_hbm, idxout_hbm, counts_hbm,
             v_buf, idx_buf, rows, zbuf, ones_buf, plane, add_sem, *sems):
    gs = sems[:RING]
    os = sems[RING:]
    cid = lax.axis_index("c")
    sid = lax.axis_index("s")
    wid = sid * 2 + cid
    base = wid * PERW

    pltpu.sync_copy(v_hbm.at[pl.ds(base, PERW)], v_buf)

    @pl.loop(jnp.int32(0), jnp.int32(zbuf.shape[0] // 16))
    def _(k):
        zbuf[pl.ds(k * 16, 16)] = jnp.zeros((16,), jnp.int32)
    for k in range(8):
        ones_buf[pl.ds(k * 16, 16)] = jnp.ones((16,), jnp.int32)
    zchunk = zbuf.shape[0]
    for k in range(PLANE // 16 // zchunk):
        pltpu.sync_copy(
            zbuf, plane.at[pl.ds(sid * (PLANE // 16) + jnp.int32(k) * zchunk,
                                 zchunk)])
    plsc.subcore_barrier()

    @pl.loop(jnp.int32(0), jnp.int32(NCH))
    def _(j):
        row = idx_buf.at[j]

        @pl.loop(jnp.int32(0), jnp.int32(CH // 16))
        def _(k):
            v = plsc.bitcast(v_buf[pl.ds(j * CH + k * 16, 16)], jnp.uint32)
            row[pl.ds(k * 16, 16)] = _remap16(v)

    pltpu.sync_copy(idx_buf, idxout_hbm.at[wid])

    _i = jnp.int32

    @pl.loop(jnp.int32(0), jnp.int32(NCH))
    def _(j):
        pltpu.async_copy(ones_buf, plane.at[idx_buf.at[_i(j)]], add_sem,
                         add=True)

    def g_desc(c, b):
        return pltpu.make_async_copy(
            table_hbm.at[idx_buf.at[_i(c)]], rows.at[_i(b)], gs[b])

    def o_desc(c, b):
        return pltpu.make_async_copy(
            rows.at[_i(b)], emb_hbm.at[pl.ds(base + _i(c) * CH, CH)], os[b])

    def issue(c, b, first):
        if not first:
            o_desc(c - RING, b).wait()
        g_desc(c, b).start()

    def work(c, b):
        g_desc(c, b).wait()
        o_desc(c, b).start()

    for b in range(RING):
        issue(b, b, True)

    @pl.loop(jnp.int32(0), jnp.int32(NCH - RING), step=jnp.int32(RING))
    def _(c0):
        for b in range(RING):
            work(c0 + b, b)
        for b in range(RING):
            issue(c0 + RING + b, b, False)

    for b in range(RING):
        work(NCH - RING + b, b)
    for b in range(RING):
        o_desc(NCH - RING + b, b).wait()

    @pl.loop(jnp.int32(0), jnp.int32(NCH))
    def _(j):
        pltpu.make_async_copy(ones_buf, plane.at[idx_buf.at[_i(j)]],
                              add_sem).wait()

    plsc.subcore_barrier()
    pltpu.sync_copy(plane.at[pl.ds(sid * (PLANE // 16), PLANE // 16)],
                    counts_hbm.at[cid].at[pl.ds(sid * (PLANE // 16),
                                                PLANE // 16)])


@functools.cache
def _make_sc_fwd():
    return pl.kernel(
        _sc_body,
        out_type=(
            jax.ShapeDtypeStruct((NVAL, EDIM), jnp.float32),
            jax.ShapeDtypeStruct((NWORK, NCH, CH), jnp.int32),
            jax.ShapeDtypeStruct((2, PLANE), jnp.int32),
        ),
        mesh=plsc.VectorSubcoreMesh(core_axis_name="c", subcore_axis_name="s"),
        compiler_params=pltpu.CompilerParams(use_tc_tiling_on_sc=False),
        scratch_types=[
            pltpu.VMEM((PERW,), jnp.int32),
            pltpu.VMEM((NCH, CH), jnp.int32),
            pltpu.VMEM((RING, CH, EDIM), jnp.float32),
            pltpu.VMEM((8192,), jnp.int32),
            pltpu.VMEM((CH,), jnp.int32),
            pltpu.VMEM_SHARED((PLANE,), jnp.int32),
        ] + [pltpu.SemaphoreType.DMA] * (2 * RING + 1),
    )


def _sc_meta_body(counts_hbm, meta_hbm, out_hbm, c0b, c1b, mb, ob, cvec):
    cid = lax.axis_index("c")
    sid = lax.axis_index("s")
    wid = sid * 2 + cid

    lanes = lax.iota(jnp.int32, 16)
    idx0 = lanes >> 1
    idx1 = idx0 + 8
    setv = (lanes & 1) ^ 1

    @pl.when(wid < BW)
    def _():
        @pl.loop(jnp.int32(0), jnp.int32(BLOOP))
        def _(i):
            sbase = wid * (BCH * BLOOP) + i * BCH
            pltpu.sync_copy(counts_hbm.at[jnp.int32(0)].at[pl.ds(sbase, BCH)],
                            c0b)
            pltpu.sync_copy(counts_hbm.at[jnp.int32(1)].at[pl.ds(sbase, BCH)],
                            c1b)
            pltpu.sync_copy(meta_hbm.at[pl.ds(sbase * 2, BCH * 2)], mb)

            @pl.loop(jnp.int32(0), jnp.int32(BCH // 16))
            def _(k):
                c = c0b[pl.ds(k * 16, 16)] + c1b[pl.ds(k * 16, 16)]
                cvec[pl.ds(jnp.int32(0), 16)] = c
                e0 = plsc.load_gather(cvec, [idx0])
                e1 = plsc.load_gather(cvec, [idx1])
                m0 = mb[pl.ds(k * 32, 16)]
                m1 = mb[pl.ds(k * 32 + 16, 16)]
                ob[pl.ds(k * 32, 16)] = jnp.where(e0 != 0, setv, m0)
                ob[pl.ds(k * 32 + 16, 16)] = jnp.where(e1 != 0, setv, m1)

            pltpu.sync_copy(ob, out_hbm.at[pl.ds(sbase * 2, BCH * 2)])


@functools.cache
def _make_sc_meta():
    return pl.kernel(
        _sc_meta_body,
        out_type=jax.ShapeDtypeStruct((2 * ZCH,), jnp.int32),
        mesh=plsc.VectorSubcoreMesh(core_axis_name="c", subcore_axis_name="s"),
        compiler_params=pltpu.CompilerParams(use_tc_tiling_on_sc=False,
                                             needs_layout_passes=False),
        scratch_types=[
            pltpu.VMEM((BCH,), jnp.int32),
            pltpu.VMEM((BCH,), jnp.int32),
            pltpu.VMEM((2 * BCH,), jnp.int32),
            pltpu.VMEM((2 * BCH,), jnp.int32),
            pltpu.VMEM((16,), jnp.int32),
        ],
    )


def kernel(values, lengths, table, metadata):
    v32 = values.astype(jnp.int32)
    emb, idxout, counts = _make_sc_fwd()(v32, table)
    meta_flat = lax.bitcast_convert_type(metadata, jnp.int32).reshape(2 * ZCH)
    new_flat = _make_sc_meta()(counts, meta_flat)
    new_metadata = lax.bitcast_convert_type(
        new_flat.reshape(ZCH, 2), jnp.int64)
    remapped = idxout.reshape(NVAL).astype(jnp.int64)
    return emb, lengths, remapped, new_metadata


# speedup vs baseline: 67.9277x; 3.2920x over previous
import functools

import jax
import jax.numpy as jnp
from jax import lax
from jax.experimental import pallas as pl
from jax.experimental.pallas import tpu as pltpu
from jax.experimental.pallas import tpu_sc as plsc

jax.config.update("jax_enable_x64", True)

ZCH = 1_000_000
PLANE = 1 << 20
EDIM = 32
NVAL = 327_680
NWORK = 32
PERW = NVAL // NWORK
CH = 128
NCH = PERW // CH
RING = 8

BW = 25
BCH = 4000
BLOOP = ZCH // (BW * BCH)

_K1_LO, _K1_HI = 0x7F4A7C15, 0x9E3779B9
_K2_LO, _K2_HI = 0x1CE4E5B9, 0xBF58476D
_M80 = (2**35 + 79) // 80
_M3125 = (2**42 + 3124) // 3125
_R32_12500 = 4796


def _u(x):
    return jnp.uint32(x)


def _mulhi(a, b_const):
    b0 = _u(b_const & 0xFFFF)
    b1 = _u(b_const >> 16)
    a0 = a & _u(0xFFFF)
    a1 = a >> _u(16)
    p00 = a0 * b0
    p01 = a0 * b1
    p10 = a1 * b0
    p11 = a1 * b1
    w = (p00 >> _u(16)) + (p01 & _u(0xFFFF)) + (p10 & _u(0xFFFF))
    return p11 + (p01 >> _u(16)) + (p10 >> _u(16)) + (w >> _u(16))


def _mod12500(x):
    q = _mulhi(x >> _u(2), _M3125) >> _u(10)
    return x - q * _u(12500)


def _remap16(v):
    qb = _mulhi(v, _M80) >> _u(3)
    bucket = v - qb * _u(80)
    lo1 = v * _u(_K1_LO)
    hi1 = _mulhi(v, _K1_LO) + v * _u(_K1_HI)
    lo2 = lo1 ^ ((hi1 << _u(3)) | (lo1 >> _u(29)))
    hi2 = hi1 ^ (hi1 >> _u(29))
    lo3 = lo2 * _u(_K2_LO)
    hi3 = _mulhi(lo2, _K2_LO) + lo2 * _u(_K2_HI) + hi2 * _u(_K2_LO)
    lo4 = lo3 ^ hi3
    s = _mod12500(_mod12500(hi3) * _u(_R32_12500) + _mod12500(lo4))
    return plsc.bitcast(bucket * _u(12500) + s, jnp.int32)


def _sc_body(v_hbm, table_hbm, emb_hbm, idxout_hbm, counts_hbm,
             v_buf, idx_buf, rows, zbuf, ones_buf, plane, add_sem, *sems):
    gs = sems[:RING]
    os = sems[RING:]
    cid = lax.axis_index("c")
    sid = lax.axis_index("s")
    wid = sid * 2 + cid
    base = wid * PERW

    pltpu.sync_copy(v_hbm.at[pl.ds(base, PERW)], v_buf)

    @pl.loop(jnp.int32(0), jnp.int32(zbuf.shape[0] // 16))
    def _(k):
        zbuf[pl.ds(k * 16, 16)] = jnp.zeros((16,), jnp.int32)
    for k in range(8):
        ones_buf[pl.ds(k * 16, 16)] = jnp.ones((16,), jnp.int32)
    zchunk = zbuf.shape[0]
    for k in range(PLANE // 16 // zchunk):
        pltpu.sync_copy(
            zbuf, plane.at[pl.ds(sid * (PLANE // 16) + jnp.int32(k) * zchunk,
                                 zchunk)])
    plsc.subcore_barrier()

    @pl.loop(jnp.int32(0), jnp.int32(NCH))
    def _(j):
        row = idx_buf.at[j]

        @pl.loop(jnp.int32(0), jnp.int32(CH // 16))
        def _(k):
            v = plsc.bitcast(v_buf[pl.ds(j * CH + k * 16, 16)], jnp.uint32)
            row[pl.ds(k * 16, 16)] = _remap16(v)

    pltpu.sync_copy(idx_buf, idxout_hbm.at[wid])

    _i = jnp.int32

    @pl.loop(jnp.int32(0), jnp.int32(NCH))
    def _(j):
        pltpu.async_copy(ones_buf, plane.at[idx_buf.at[_i(j)]], add_sem,
                         add=True)

    def g_desc(c, b):
        return pltpu.make_async_copy(
            table_hbm.at[idx_buf.at[_i(c)]], rows.at[_i(b)], gs[b])

    def o_desc(c, b):
        return pltpu.make_async_copy(
            rows.at[_i(b)], emb_hbm.at[pl.ds(base + _i(c) * CH, CH)], os[b])

    def issue(c, b, first):
        if not first:
            o_desc(c - RING, b).wait()
        g_desc(c, b).start()

    def work(c, b):
        g_desc(c, b).wait()
        o_desc(c, b).start()

    for b in range(RING):
        issue(b, b, True)

    @pl.loop(jnp.int32(0), jnp.int32(NCH - RING), step=jnp.int32(RING))
    def _(c0):
        for b in range(RING):
            work(c0 + b, b)
        for b in range(RING):
            issue(c0 + RING + b, b, False)

    for b in range(RING):
        work(NCH - RING + b, b)
    for b in range(RING):
        o_desc(NCH - RING + b, b).wait()

    @pl.loop(jnp.int32(0), jnp.int32(NCH))
    def _(j):
        pltpu.make_async_copy(ones_buf, plane.at[idx_buf.at[_i(j)]],
                              add_sem).wait()

    plsc.subcore_barrier()
    pltpu.sync_copy(plane.at[pl.ds(sid * (PLANE // 16), PLANE // 16)],
                    counts_hbm.at[cid].at[pl.ds(sid * (PLANE // 16),
                                                PLANE // 16)])


@functools.cache
def _make_sc_fwd():
    return pl.kernel(
        _sc_body,
        out_type=(
            jax.ShapeDtypeStruct((NVAL, EDIM), jnp.float32),
            jax.ShapeDtypeStruct((NWORK, NCH, CH), jnp.int32),
            jax.ShapeDtypeStruct((2, PLANE), jnp.int32),
        ),
        mesh=plsc.VectorSubcoreMesh(core_axis_name="c", subcore_axis_name="s"),
        compiler_params=pltpu.CompilerParams(use_tc_tiling_on_sc=False),
        scratch_types=[
            pltpu.VMEM((PERW,), jnp.int32),
            pltpu.VMEM((NCH, CH), jnp.int32),
            pltpu.VMEM((RING, CH, EDIM), jnp.float32),
            pltpu.VMEM((8192,), jnp.int32),
            pltpu.VMEM((CH,), jnp.int32),
            pltpu.VMEM_SHARED((PLANE,), jnp.int32),
        ] + [pltpu.SemaphoreType.DMA] * (2 * RING + 1),
    )


def _sc_meta_body(counts_hbm, mlo_hbm, mhi_hbm, olo_hbm, ohi_hbm,
                  c0b, c1b, mlob, mhib, olob, ohib):
    cid = lax.axis_index("c")
    sid = lax.axis_index("s")
    wid = sid * 2 + cid

    @pl.when(wid < BW)
    def _():
        @pl.loop(jnp.int32(0), jnp.int32(BLOOP))
        def _(i):
            sbase = wid * (BCH * BLOOP) + i * BCH
            pltpu.sync_copy(counts_hbm.at[jnp.int32(0)].at[pl.ds(sbase, BCH)],
                            c0b)
            pltpu.sync_copy(counts_hbm.at[jnp.int32(1)].at[pl.ds(sbase, BCH)],
                            c1b)
            pltpu.sync_copy(mlo_hbm.at[pl.ds(sbase, BCH)], mlob)
            pltpu.sync_copy(mhi_hbm.at[pl.ds(sbase, BCH)], mhib)

            @pl.loop(jnp.int32(0), jnp.int32(BCH // 16))
            def _(k):
                t = (c0b[pl.ds(k * 16, 16)] + c1b[pl.ds(k * 16, 16)]) != 0
                one = jnp.full((16,), 1, jnp.int32)
                zero = jnp.zeros((16,), jnp.int32)
                olob[pl.ds(k * 16, 16)] = jnp.where(
                    t, one, mlob[pl.ds(k * 16, 16)])
                ohib[pl.ds(k * 16, 16)] = jnp.where(
                    t, zero, mhib[pl.ds(k * 16, 16)])

            pltpu.sync_copy(olob, olo_hbm.at[pl.ds(sbase, BCH)])
            pltpu.sync_copy(ohib, ohi_hbm.at[pl.ds(sbase, BCH)])


@functools.cache
def _make_sc_meta():
    return pl.kernel(
        _sc_meta_body,
        out_type=(
            jax.ShapeDtypeStruct((ZCH,), jnp.int32),
            jax.ShapeDtypeStruct((ZCH,), jnp.int32),
        ),
        mesh=plsc.VectorSubcoreMesh(core_axis_name="c", subcore_axis_name="s"),
        compiler_params=pltpu.CompilerParams(use_tc_tiling_on_sc=False,
                                             needs_layout_passes=False),
        scratch_types=[pltpu.VMEM((BCH,), jnp.int32)] * 6,
    )


def kernel(values, lengths, table, metadata):
    v32 = values.astype(jnp.int32)
    emb, idxout, counts = _make_sc_fwd()(v32, table)
    mlo = metadata.astype(jnp.int32)
    mhi = (metadata >> jnp.int64(32)).astype(jnp.int32)
    olo, ohi = _make_sc_meta()(counts, mlo, mhi)
    new_metadata = (
        (ohi.astype(jnp.int64) << jnp.int64(32))
        | olo.astype(jnp.uint32).astype(jnp.int64))
    remapped = idxout.reshape(NVAL).astype(jnp.int64)
    return emb, lengths, remapped, new_metadata
